# baseline (speedup 1.0000x reference)
import functools

import jax
import jax.numpy as jnp
from jax import lax
from jax.experimental import pallas as pl
from jax.experimental.pallas import tpu as pltpu
from jax.experimental.pallas import tpu_sc as plsc

N_TOK = 16384
DIM = 256
MAX_N = 2048
NB = 16
BLK = 256
KMAX = N_TOK // BLK
IBLK = 512
JBLK = 2048
XS_ROWS = N_TOK + 2 * BLK
EPS = 1e-5


def _ln_tanh(h, g, beta):
    mu = jnp.mean(h, axis=-1, keepdims=True)
    var = jnp.mean((h - mu) ** 2, axis=-1, keepdims=True)
    return jnp.tanh((h - mu) * lax.rsqrt(var + EPS) * g + beta)


def _prep_kernel(x_ref, bR_ref, rw_ref, rb_ref,
                 w1p_ref, b1_ref, g_ref, be_ref, w2_ref, b2_ref,
                 mag_ref, n_ref, st_ref, adj_ref, tab_ref):
    mag_ref[...] = jnp.dot(x_ref[...], rw_ref[...],
                           preferred_element_type=jnp.float32) + rb_ref[...]
    segs = lax.broadcasted_iota(jnp.int32, (NB, 1), 0)
    bj = bR_ref[...]
    n_ref[...] = jnp.sum((bj == segs).astype(jnp.float32), axis=1,
                         keepdims=True).astype(jnp.int32)
    nf = jnp.sum((bj == segs).astype(jnp.float32), axis=1, keepdims=True)
    starts = jnp.sum((bj < segs).astype(jnp.float32), axis=1, keepdims=True)
    pad_n = jnp.ceil(nf / 8.0) * 8.0
    segs_r = lax.broadcasted_iota(jnp.int32, (1, NB), 1)
    tri = (segs > segs_r).astype(jnp.float32)
    astart = jnp.dot(tri, pad_n, preferred_element_type=jnp.float32)
    st_ref[...] = astart.astype(jnp.int32)
    adj_ref[...] = astart - starts
    h = _ln_tanh(w1p_ref[...] + b1_ref[...], g_ref[...], be_ref[...])
    tab_ref[...] = jnp.dot(h, w2_ref[...],
                           preferred_element_type=jnp.float32) + b2_ref[...]


def _prep_call(x, batch_row, rank_W, rank_b, w1pad, p):
    hid = w1pad.shape[1]
    return pl.pallas_call(
        _prep_kernel,
        out_shape=(
            jax.ShapeDtypeStruct((N_TOK, 1), jnp.float32),
            jax.ShapeDtypeStruct((NB, 1), jnp.int32),
            jax.ShapeDtypeStruct((NB, 1), jnp.int32),
            jax.ShapeDtypeStruct((NB, 1), jnp.float32),
            jax.ShapeDtypeStruct((MAX_N + BLK, DIM), jnp.float32),
        ),
    )(x, batch_row, rank_W, rank_b.reshape(1, 1),
      w1pad, p["b1"].reshape(1, hid), p["g"].reshape(1, hid),
      p["beta"].reshape(1, hid), p["W2"], p["b2"].reshape(1, DIM))


def _rank_kernel(magC_ref, magR_ref, bC_ref, bR_ref, adj_ref, dst_ref):
    i = pl.program_id(0)
    i0 = pl.multiple_of(i * IBLK, IBLK)
    mi = magC_ref[pl.ds(i0, IBLK), :]
    bi = bC_ref[pl.ds(i0, IBLK), :]
    ii = i0 + lax.broadcasted_iota(jnp.int32, (IBLK, 1), 0)
    acc = jnp.zeros((IBLK, 1), jnp.float32)
    for c in range(N_TOK // JBLK):
        j0 = c * JBLK
        mj = magR_ref[:, pl.ds(j0, JBLK)]
        bj = bR_ref[:, pl.ds(j0, JBLK)]
        jj = j0 + lax.broadcasted_iota(jnp.int32, (1, JBLK), 1)
        cond = (bj < bi) | ((bj == bi) &
                            ((mj < mi) | ((mj == mi) & (jj < ii))))
        acc = acc + jnp.sum(cond.astype(jnp.float32), axis=1, keepdims=True)
    segs_r = lax.broadcasted_iota(jnp.int32, (1, NB), 1)
    oh = (bi == segs_r).astype(jnp.float32)
    acc = acc + jnp.dot(oh, adj_ref[...], preferred_element_type=jnp.float32)
    dst_ref[...] = acc.astype(jnp.int32)


def _rank_call(mag_col, mag_row, batch_col, batch_row, adj):
    full = lambda i: (0, 0)
    return pl.pallas_call(
        _rank_kernel,
        grid=(N_TOK // IBLK,),
        in_specs=[
            pl.BlockSpec((N_TOK, 1), full),
            pl.BlockSpec((1, N_TOK), full),
            pl.BlockSpec((N_TOK, 1), full),
            pl.BlockSpec((1, N_TOK), full),
            pl.BlockSpec((NB, 1), full),
        ],
        out_specs=pl.BlockSpec((IBLK, 1), lambda i: (i, 0)),
        out_shape=jax.ShapeDtypeStruct((N_TOK, 1), jnp.int32),
    )(mag_col, mag_row, batch_col, batch_row, adj)


_SC_CHUNK = 128


def _sc_scatter(x, dst):
    info = plsc.get_sparse_core_info()
    nc, ns = info.num_cores, info.num_subcores
    nw = nc * ns
    rows_per_w = N_TOK // nw
    nchunk = rows_per_w // _SC_CHUNK
    mesh = plsc.VectorSubcoreMesh(core_axis_name="c", subcore_axis_name="s")

    @functools.partial(
        pl.kernel, mesh=mesh,
        out_type=jax.ShapeDtypeStruct((XS_ROWS, DIM), jnp.float32),
        scratch_types=[
            pltpu.VMEM((_SC_CHUNK,), jnp.int32),
            pltpu.VMEM((_SC_CHUNK, DIM), jnp.float32),
            pltpu.SemaphoreType.DMA,
        ],
    )
    def scatter(x_hbm, dst_hbm, out_hbm, idx_v, rows_v, sem):
        wid = lax.axis_index("s") * nc + lax.axis_index("c")
        for c in range(nchunk):
            base = wid * rows_per_w + c * _SC_CHUNK
            pltpu.sync_copy(dst_hbm.at[pl.ds(base, _SC_CHUNK)], idx_v)
            pltpu.sync_copy(x_hbm.at[pl.ds(base, _SC_CHUNK)], rows_v)
            pltpu.async_copy(rows_v, out_hbm.at[idx_v], sem).wait()

    return scatter(x, dst)


def _ds_kernel(n_s, st_s, xs_ref, tab_ref, w1_ref, b1_ref, g_ref, be_ref,
               w2_ref, b2_ref, z_ref):
    b = pl.program_id(0)
    k = pl.program_id(1)
    nb = n_s[b]

    @pl.when(k == 0)
    def _init():
        z_ref[...] = jnp.zeros_like(z_ref)

    @pl.when(k * BLK < nb)
    def _work():
        tok0 = pl.multiple_of(st_s[b] + k * BLK, 8)
        xb = xs_ref[pl.ds(tok0, BLK), :]
        h = jnp.dot(xb, w1_ref[...], preferred_element_type=jnp.float32) + b1_ref[...]
        v = jnp.dot(_ln_tanh(h, g_ref[...], be_ref[...]), w2_ref[...],
                    preferred_element_type=jnp.float32) + b2_ref[...]
        t0 = pl.multiple_of(jnp.minimum(k * BLK, MAX_N), BLK)
        tb = tab_ref[pl.ds(t0, BLK), :]
        rows = lax.broadcasted_iota(jnp.int32, (BLK, 1), 0)
        y1 = jnp.where(rows < nb - k * BLK, v * tb, 0.0)
        z_ref[...] += jnp.sum(y1, axis=0, keepdims=True)[None]


def _ds_call(n16, st16, xs, table, p):
    full = lambda *_: (0, 0)
    grid_spec = pltpu.PrefetchScalarGridSpec(
        num_scalar_prefetch=2,
        grid=(NB, KMAX),
        in_specs=[
            pl.BlockSpec((XS_ROWS, DIM), full),
            pl.BlockSpec((MAX_N + BLK, DIM), full),
            pl.BlockSpec((DIM, DIM), full),
            pl.BlockSpec((1, DIM), full),
            pl.BlockSpec((1, DIM), full),
            pl.BlockSpec((1, DIM), full),
            pl.BlockSpec((DIM, DIM), full),
            pl.BlockSpec((1, DIM), full),
        ],
        out_specs=pl.BlockSpec((1, 1, DIM), lambda b, k, *_: (b, 0, 0)),
    )
    return pl.pallas_call(
        _ds_kernel, grid_spec=grid_spec,
        out_shape=jax.ShapeDtypeStruct((NB, 1, DIM), jnp.float32),
    )(n16, st16, xs, table, p["W1"], p["b1"].reshape(1, DIM),
      p["g"].reshape(1, DIM), p["beta"].reshape(1, DIM), p["W2"],
      p["b2"].reshape(1, DIM))


HID_V = 384
HID_K = 1280


def _main_kernel(n_s, st_s, xs_ref, zds_ref,
                 w1vt_ref, w1vb_ref, b1v_ref, gv_ref, bev_ref, w2v_ref, b2v_ref,
                 w1kp_ref, w1kb_ref, b1k_ref, gk_ref, bek_ref, w2k_ref, b2k_ref,
                 cw_ref, cb_ref, out_ref, zv_s, zk_s):
    b = pl.program_id(0)
    k = pl.program_id(1)
    nb = n_s[b]

    @pl.when(k == 0)
    def _init():
        segs_r = lax.broadcasted_iota(jnp.int32, (1, NB), 1)
        zrow = jnp.dot((segs_r == b).astype(jnp.float32), zds_ref[...],
                       preferred_element_type=jnp.float32)
        zv_s[...] = jnp.dot(zrow, w1vb_ref[...], preferred_element_type=jnp.float32)
        zk_s[...] = jnp.dot(zrow, w1kb_ref[...], preferred_element_type=jnp.float32)
        out_ref[...] = (nb.astype(jnp.float32) * cw_ref[...] + cb_ref[...])[None]

    @pl.when(k * BLK < nb)
    def _work():
        tok0 = pl.multiple_of(st_s[b] + k * BLK, 8)
        xb = xs_ref[pl.ds(tok0, BLK), :]
        hv = (jnp.dot(xb, w1vt_ref[...], preferred_element_type=jnp.float32)
              + zv_s[...] + b1v_ref[...])
        val = jnp.dot(_ln_tanh(hv, gv_ref[...], bev_ref[...]), w2v_ref[...],
                      preferred_element_type=jnp.float32) + b2v_ref[...]
        t0 = pl.multiple_of(jnp.minimum(k * BLK, MAX_N), BLK)
        hk = w1kp_ref[pl.ds(t0, BLK), :] + zk_s[...] + b1k_ref[...]
        key = jnp.dot(_ln_tanh(hk, gk_ref[...], bek_ref[...]), w2k_ref[...],
                      preferred_element_type=jnp.float32) + b2k_ref[...]
        rows = lax.broadcasted_iota(jnp.int32, (BLK, 1), 0)
        y = jnp.where(rows < nb - k * BLK, val * key, 0.0)
        out_ref[...] += jnp.sum(y, axis=0, keepdims=True)[None]


def _main_call(n16, st16, xs, z_ds, w1kpad, pv, pk, card_W, card_b):
    full = lambda *_: (0, 0)
    grid_spec = pltpu.PrefetchScalarGridSpec(
        num_scalar_prefetch=2,
        grid=(NB, KMAX),
        in_specs=[
            pl.BlockSpec((XS_ROWS, DIM), full),
            pl.BlockSpec((NB, DIM), full),
            pl.BlockSpec((DIM, HID_V), full),
            pl.BlockSpec((DIM, HID_V), full),
            pl.BlockSpec((1, HID_V), full),
            pl.BlockSpec((1, HID_V), full),
            pl.BlockSpec((1, HID_V), full),
            pl.BlockSpec((HID_V, DIM), full),
            pl.BlockSpec((1, DIM), full),
            pl.BlockSpec((MAX_N + BLK, HID_K), full),
            pl.BlockSpec((DIM, HID_K), full),
            pl.BlockSpec((1, HID_K), full),
            pl.BlockSpec((1, HID_K), full),
            pl.BlockSpec((1, HID_K), full),
            pl.BlockSpec((HID_K, DIM), full),
            pl.BlockSpec((1, DIM), full),
            pl.BlockSpec((1, DIM), full),
            pl.BlockSpec((1, DIM), full),
        ],
        out_specs=pl.BlockSpec((1, 1, DIM), lambda b, k, *_: (b, 0, 0)),
        scratch_shapes=[
            pltpu.VMEM((1, HID_V), jnp.float32),
            pltpu.VMEM((1, HID_K), jnp.float32),
        ],
    )
    return pl.pallas_call(
        _main_kernel, grid_spec=grid_spec,
        out_shape=jax.ShapeDtypeStruct((NB, 1, DIM), jnp.float32),
    )(n16, st16, xs, z_ds,
      pv["W1"][:DIM], pv["W1"][DIM:], pv["b1"].reshape(1, HID_V),
      pv["g"].reshape(1, HID_V), pv["beta"].reshape(1, HID_V), pv["W2"],
      pv["b2"].reshape(1, DIM),
      w1kpad, pk["W1"][MAX_N:], pk["b1"].reshape(1, HID_K),
      pk["g"].reshape(1, HID_K), pk["beta"].reshape(1, HID_K), pk["W2"],
      pk["b2"].reshape(1, DIM),
      card_W, card_b.reshape(1, DIM))


def kernel(x, batch, n_batches, params):
    del n_batches
    batch = batch.astype(jnp.int32)
    batch_col = batch.reshape(N_TOK, 1)
    batch_row = batch.reshape(1, N_TOK)

    pds = params["key_ds"]
    hid_ds = pds["W1"].shape[1]
    w1ds_pad = jnp.concatenate(
        [pds["W1"], jnp.zeros((BLK, hid_ds), jnp.float32)], axis=0)
    mag, n16, st16, adj, table = _prep_call(
        x, batch_row, params["rank_W"], params["rank_b"], w1ds_pad, pds)

    dst = _rank_call(mag, mag.reshape(1, N_TOK), batch_col, batch_row, adj)
    xs = _sc_scatter(x, dst.reshape(N_TOK))

    n16 = n16.reshape(NB)
    st16 = st16.reshape(NB)
    z_ds = _ds_call(n16, st16, xs, table, params["val_ds"]).reshape(NB, DIM)

    pk = params["key_main"]
    w1k_pad = jnp.concatenate(
        [pk["W1"][:MAX_N], jnp.zeros((BLK, HID_K), jnp.float32)], axis=0)
    out = _main_call(n16, st16, xs, z_ds, w1k_pad,
                     params["val_main"], pk, params["card_W"],
                     params["card_b"])
    return out.reshape(NB, DIM)


# speedup vs baseline: 3.2609x; 3.2609x over previous
import functools

import jax
import jax.numpy as jnp
from jax import lax
from jax.experimental import pallas as pl
from jax.experimental.pallas import tpu as pltpu
from jax.experimental.pallas import tpu_sc as plsc

N_TOK = 16384
DIM = 256
MAX_N = 2048
NB = 16
BLK = 256
KMAX = N_TOK // BLK
IBLK = 512
JBLK = 2048
XS_ROWS = N_TOK + 2 * BLK
WMAX = NB + KMAX
JR = 512
NIB = N_TOK // IBLK
EPS = 1e-5


def _ln_tanh(h, g, beta):
    mu = jnp.mean(h, axis=-1, keepdims=True)
    var = jnp.mean((h - mu) ** 2, axis=-1, keepdims=True)
    return jnp.tanh((h - mu) * lax.rsqrt(var + EPS) * g + beta)


def _prep_kernel(x_ref, bR_ref, rw_ref, rb_ref,
                 w1p_ref, b1_ref, g_ref, be_ref, w2_ref, b2_ref,
                 mag_ref, n_ref, st_ref, adj_ref, wseg_ref, wblk_ref,
                 jlo_ref, jhi_ref, tab_ref):
    mag_ref[...] = jnp.dot(x_ref[...], rw_ref[...],
                           preferred_element_type=jnp.float32) + rb_ref[...]
    segs = lax.broadcasted_iota(jnp.int32, (NB, 1), 0)
    bj = bR_ref[...]
    n_ref[...] = jnp.sum((bj == segs).astype(jnp.float32), axis=1,
                         keepdims=True).astype(jnp.int32)
    nf = jnp.sum((bj == segs).astype(jnp.float32), axis=1, keepdims=True)
    starts = jnp.sum((bj < segs).astype(jnp.float32), axis=1, keepdims=True)
    pad_n = jnp.ceil(nf / 8.0) * 8.0
    segs_r = lax.broadcasted_iota(jnp.int32, (1, NB), 1)
    tri = (segs > segs_r).astype(jnp.float32)
    astart = jnp.dot(tri, pad_n, preferred_element_type=jnp.float32)
    st_ref[...] = astart.astype(jnp.int32)
    adj_ref[...] = astart - starts
    nblk = jnp.maximum(jnp.ceil(nf / BLK), 1.0)
    incl = (segs >= segs_r).astype(jnp.float32)
    cum = jnp.dot(incl, nblk, preferred_element_type=jnp.float32)
    w_row = lax.broadcasted_iota(jnp.int32, (1, WMAX), 1).astype(jnp.float32)
    wseg = jnp.sum((cum <= w_row).astype(jnp.float32), axis=0,
                   keepdims=True)
    oh_w = (segs.astype(jnp.float32) == wseg).astype(jnp.float32)
    cumexc = cum - nblk
    wblk = w_row - jnp.sum(oh_w * cumexc, axis=0, keepdims=True)
    wseg_ref[...] = wseg.astype(jnp.int32)
    wblk_ref[...] = wblk.astype(jnp.int32)
    i0_row = lax.broadcasted_iota(jnp.int32, (1, NIB), 1).astype(jnp.float32) * IBLK
    lo_tok = jnp.max(jnp.where(starts <= i0_row, starts, 0.0), axis=0,
                     keepdims=True)
    ends = starts + nf
    hi_tok = jnp.min(jnp.where(ends >= i0_row + IBLK, ends, float(N_TOK)),
                     axis=0, keepdims=True)
    jlo_ref[...] = jnp.floor(lo_tok / JR).astype(jnp.int32)
    jhi_ref[...] = jnp.ceil(hi_tok / JR).astype(jnp.int32)
    h = _ln_tanh(w1p_ref[...] + b1_ref[...], g_ref[...], be_ref[...])
    tab_ref[...] = jnp.dot(h, w2_ref[...],
                           preferred_element_type=jnp.float32) + b2_ref[...]


def _prep_call(x, batch_row, rank_W, rank_b, w1pad, p):
    hid = w1pad.shape[1]
    return pl.pallas_call(
        _prep_kernel,
        out_shape=(
            jax.ShapeDtypeStruct((N_TOK, 1), jnp.float32),
            jax.ShapeDtypeStruct((NB, 1), jnp.int32),
            jax.ShapeDtypeStruct((NB, 1), jnp.int32),
            jax.ShapeDtypeStruct((NB, 1), jnp.float32),
            jax.ShapeDtypeStruct((1, WMAX), jnp.int32),
            jax.ShapeDtypeStruct((1, WMAX), jnp.int32),
            jax.ShapeDtypeStruct((1, NIB), jnp.int32),
            jax.ShapeDtypeStruct((1, NIB), jnp.int32),
            jax.ShapeDtypeStruct((MAX_N + BLK, DIM), jnp.float32),
        ),
    )(x, batch_row, rank_W, rank_b.reshape(1, 1),
      w1pad, p["b1"].reshape(1, hid), p["g"].reshape(1, hid),
      p["beta"].reshape(1, hid), p["W2"], p["b2"].reshape(1, DIM))


def _rank_kernel(jlo_s, jhi_s, magC_ref, magR_ref, bC_ref, bR_ref,
                 adj_ref, dst_ref):
    i = pl.program_id(0)
    i0 = pl.multiple_of(i * IBLK, IBLK)
    mi = magC_ref[pl.ds(i0, IBLK), :]
    bi = bC_ref[pl.ds(i0, IBLK), :]
    ii = i0 + lax.broadcasted_iota(jnp.int32, (IBLK, 1), 0)
    lo = jlo_s[i]
    acc = jnp.full((IBLK, 1), 0.0, jnp.float32) + (lo * JR).astype(jnp.float32)

    def body(c, acc):
        j0 = pl.multiple_of(c * JR, JR)
        mj = magR_ref[:, pl.ds(j0, JR)]
        bj = bR_ref[:, pl.ds(j0, JR)]
        jj = j0 + lax.broadcasted_iota(jnp.int32, (1, JR), 1)
        cond = (bj < bi) | ((bj == bi) &
                            ((mj < mi) | ((mj == mi) & (jj < ii))))
        return acc + jnp.sum(cond.astype(jnp.float32), axis=1, keepdims=True)

    acc = lax.fori_loop(lo, jhi_s[i], body, acc)
    segs_r = lax.broadcasted_iota(jnp.int32, (1, NB), 1)
    oh = (bi == segs_r).astype(jnp.float32)
    acc = acc + jnp.dot(oh, adj_ref[...], preferred_element_type=jnp.float32)
    dst_ref[...] = acc.astype(jnp.int32)


def _rank_call(jlo, jhi, mag_col, mag_row, batch_col, batch_row, adj):
    full = lambda *_: (0, 0)
    grid_spec = pltpu.PrefetchScalarGridSpec(
        num_scalar_prefetch=2,
        grid=(NIB,),
        in_specs=[
            pl.BlockSpec((N_TOK, 1), full),
            pl.BlockSpec((1, N_TOK), full),
            pl.BlockSpec((N_TOK, 1), full),
            pl.BlockSpec((1, N_TOK), full),
            pl.BlockSpec((NB, 1), full),
        ],
        out_specs=pl.BlockSpec((IBLK, 1), lambda i, *_: (i, 0)),
    )
    return pl.pallas_call(
        _rank_kernel, grid_spec=grid_spec,
        out_shape=jax.ShapeDtypeStruct((N_TOK, 1), jnp.int32),
    )(jlo, jhi, mag_col, mag_row, batch_col, batch_row, adj)


_SC_CHUNK = 128


def _sc_scatter(x, dst):
    info = plsc.get_sparse_core_info()
    nc, ns = info.num_cores, info.num_subcores
    nw = nc * ns
    rows_per_w = N_TOK // nw
    nchunk = rows_per_w // _SC_CHUNK
    mesh = plsc.VectorSubcoreMesh(core_axis_name="c", subcore_axis_name="s")

    @functools.partial(
        pl.kernel, mesh=mesh,
        out_type=jax.ShapeDtypeStruct((XS_ROWS, DIM), jnp.float32),
        scratch_types=[
            pltpu.VMEM((_SC_CHUNK,), jnp.int32),
            pltpu.VMEM((_SC_CHUNK, DIM), jnp.float32),
            pltpu.SemaphoreType.DMA,
        ],
    )
    def scatter(x_hbm, dst_hbm, out_hbm, idx_v, rows_v, sem):
        wid = lax.axis_index("s") * nc + lax.axis_index("c")
        for c in range(nchunk):
            base = wid * rows_per_w + c * _SC_CHUNK
            pltpu.sync_copy(dst_hbm.at[pl.ds(base, _SC_CHUNK)], idx_v)
            pltpu.sync_copy(x_hbm.at[pl.ds(base, _SC_CHUNK)], rows_v)
            pltpu.async_copy(rows_v, out_hbm.at[idx_v], sem).wait()

    return scatter(x, dst)


def _ds_kernel(n_s, st_s, wseg_s, wblk_s, xs_ref, tab_ref, w1_ref, b1_ref,
               g_ref, be_ref, w2_ref, b2_ref, z_ref):
    w = pl.program_id(0)
    b = wseg_s[w]
    k = wblk_s[w]
    valid = b < NB
    nb = n_s[jnp.minimum(b, NB - 1)]

    @pl.when(valid & (k == 0))
    def _init():
        z_ref[...] = jnp.zeros_like(z_ref)

    @pl.when(valid & (k * BLK < nb))
    def _work():
        tok0 = pl.multiple_of(st_s[jnp.minimum(b, NB - 1)] + k * BLK, 8)
        xb = xs_ref[pl.ds(tok0, BLK), :]
        h = jnp.dot(xb, w1_ref[...], preferred_element_type=jnp.float32) + b1_ref[...]
        v = jnp.dot(_ln_tanh(h, g_ref[...], be_ref[...]), w2_ref[...],
                    preferred_element_type=jnp.float32) + b2_ref[...]
        t0 = pl.multiple_of(jnp.minimum(k * BLK, MAX_N), BLK)
        tb = tab_ref[pl.ds(t0, BLK), :]
        rows = lax.broadcasted_iota(jnp.int32, (BLK, 1), 0)
        y1 = jnp.where(rows < nb - k * BLK, v * tb, 0.0)
        z_ref[...] += jnp.sum(y1, axis=0, keepdims=True)[None]


def _ds_call(n16, st16, wseg, wblk, xs, table, p):
    full = lambda *_: (0, 0)
    grid_spec = pltpu.PrefetchScalarGridSpec(
        num_scalar_prefetch=4,
        grid=(WMAX,),
        in_specs=[
            pl.BlockSpec((XS_ROWS, DIM), full),
            pl.BlockSpec((MAX_N + BLK, DIM), full),
            pl.BlockSpec((DIM, DIM), full),
            pl.BlockSpec((1, DIM), full),
            pl.BlockSpec((1, DIM), full),
            pl.BlockSpec((1, DIM), full),
            pl.BlockSpec((DIM, DIM), full),
            pl.BlockSpec((1, DIM), full),
        ],
        out_specs=pl.BlockSpec(
            (1, 1, DIM),
            lambda w, n_s, st_s, wseg_s, wblk_s: (
                jnp.minimum(wseg_s[w], NB - 1), 0, 0)),
    )
    return pl.pallas_call(
        _ds_kernel, grid_spec=grid_spec,
        out_shape=jax.ShapeDtypeStruct((NB, 1, DIM), jnp.float32),
    )(n16, st16, wseg, wblk, xs, table, p["W1"], p["b1"].reshape(1, DIM),
      p["g"].reshape(1, DIM), p["beta"].reshape(1, DIM), p["W2"],
      p["b2"].reshape(1, DIM))


HID_V = 384
HID_K = 1280


def _main_kernel(n_s, st_s, wseg_s, wblk_s, xs_ref, zds_ref,
                 w1vt_ref, w1vb_ref, b1v_ref, gv_ref, bev_ref, w2v_ref, b2v_ref,
                 w1kp_ref, w1kb_ref, b1k_ref, gk_ref, bek_ref, w2k_ref, b2k_ref,
                 cw_ref, cb_ref, out_ref, zv_s, zk_s):
    w = pl.program_id(0)
    b = wseg_s[w]
    k = wblk_s[w]
    valid = b < NB
    nb = n_s[jnp.minimum(b, NB - 1)]

    @pl.when(valid & (k == 0))
    def _init():
        segs_r = lax.broadcasted_iota(jnp.int32, (1, NB), 1)
        zrow = jnp.dot((segs_r == b).astype(jnp.float32), zds_ref[...],
                       preferred_element_type=jnp.float32)
        zv_s[...] = jnp.dot(zrow, w1vb_ref[...], preferred_element_type=jnp.float32)
        zk_s[...] = jnp.dot(zrow, w1kb_ref[...], preferred_element_type=jnp.float32)
        out_ref[...] = (nb.astype(jnp.float32) * cw_ref[...] + cb_ref[...])[None]

    @pl.when(valid & (k * BLK < nb))
    def _work():
        tok0 = pl.multiple_of(st_s[jnp.minimum(b, NB - 1)] + k * BLK, 8)
        xb = xs_ref[pl.ds(tok0, BLK), :]
        hv = (jnp.dot(xb, w1vt_ref[...], preferred_element_type=jnp.float32)
              + zv_s[...] + b1v_ref[...])
        val = jnp.dot(_ln_tanh(hv, gv_ref[...], bev_ref[...]), w2v_ref[...],
                      preferred_element_type=jnp.float32) + b2v_ref[...]
        t0 = pl.multiple_of(jnp.minimum(k * BLK, MAX_N), BLK)
        hk = w1kp_ref[pl.ds(t0, BLK), :] + zk_s[...] + b1k_ref[...]
        key = jnp.dot(_ln_tanh(hk, gk_ref[...], bek_ref[...]), w2k_ref[...],
                      preferred_element_type=jnp.float32) + b2k_ref[...]
        rows = lax.broadcasted_iota(jnp.int32, (BLK, 1), 0)
        y = jnp.where(rows < nb - k * BLK, val * key, 0.0)
        out_ref[...] += jnp.sum(y, axis=0, keepdims=True)[None]


def _main_call(n16, st16, wseg, wblk, xs, z_ds, w1kpad, pv, pk, card_W,
               card_b):
    full = lambda *_: (0, 0)
    grid_spec = pltpu.PrefetchScalarGridSpec(
        num_scalar_prefetch=4,
        grid=(WMAX,),
        in_specs=[
            pl.BlockSpec((XS_ROWS, DIM), full),
            pl.BlockSpec((NB, DIM), full),
            pl.BlockSpec((DIM, HID_V), full),
            pl.BlockSpec((DIM, HID_V), full),
            pl.BlockSpec((1, HID_V), full),
            pl.BlockSpec((1, HID_V), full),
            pl.BlockSpec((1, HID_V), full),
            pl.BlockSpec((HID_V, DIM), full),
            pl.BlockSpec((1, DIM), full),
            pl.BlockSpec((MAX_N + BLK, HID_K), full),
            pl.BlockSpec((DIM, HID_K), full),
            pl.BlockSpec((1, HID_K), full),
            pl.BlockSpec((1, HID_K), full),
            pl.BlockSpec((1, HID_K), full),
            pl.BlockSpec((HID_K, DIM), full),
            pl.BlockSpec((1, DIM), full),
            pl.BlockSpec((1, DIM), full),
            pl.BlockSpec((1, DIM), full),
        ],
        out_specs=pl.BlockSpec(
            (1, 1, DIM),
            lambda w, n_s, st_s, wseg_s, wblk_s: (
                jnp.minimum(wseg_s[w], NB - 1), 0, 0)),
        scratch_shapes=[
            pltpu.VMEM((1, HID_V), jnp.float32),
            pltpu.VMEM((1, HID_K), jnp.float32),
        ],
    )
    return pl.pallas_call(
        _main_kernel, grid_spec=grid_spec,
        out_shape=jax.ShapeDtypeStruct((NB, 1, DIM), jnp.float32),
    )(n16, st16, wseg, wblk, xs, z_ds,
      pv["W1"][:DIM], pv["W1"][DIM:], pv["b1"].reshape(1, HID_V),
      pv["g"].reshape(1, HID_V), pv["beta"].reshape(1, HID_V), pv["W2"],
      pv["b2"].reshape(1, DIM),
      w1kpad, pk["W1"][MAX_N:], pk["b1"].reshape(1, HID_K),
      pk["g"].reshape(1, HID_K), pk["beta"].reshape(1, HID_K), pk["W2"],
      pk["b2"].reshape(1, DIM),
      card_W, card_b.reshape(1, DIM))


def kernel(x, batch, n_batches, params):
    del n_batches
    batch = batch.astype(jnp.int32)
    batch_col = batch.reshape(N_TOK, 1)
    batch_row = batch.reshape(1, N_TOK)

    pds = params["key_ds"]
    hid_ds = pds["W1"].shape[1]
    w1ds_pad = jnp.concatenate(
        [pds["W1"], jnp.zeros((BLK, hid_ds), jnp.float32)], axis=0)
    mag, n16, st16, adj, wseg, wblk, jlo, jhi, table = _prep_call(
        x, batch_row, params["rank_W"], params["rank_b"], w1ds_pad, pds)

    dst = _rank_call(jlo.reshape(NIB), jhi.reshape(NIB), mag,
                     mag.reshape(1, N_TOK), batch_col, batch_row, adj)
    xs = _sc_scatter(x, dst.reshape(N_TOK))

    n16 = n16.reshape(NB)
    st16 = st16.reshape(NB)
    wseg = wseg.reshape(WMAX)
    wblk = wblk.reshape(WMAX)
    z_ds = _ds_call(n16, st16, wseg, wblk, xs, table,
                    params["val_ds"]).reshape(NB, DIM)

    pk = params["key_main"]
    w1k_pad = jnp.concatenate(
        [pk["W1"][:MAX_N], jnp.zeros((BLK, HID_K), jnp.float32)], axis=0)
    out = _main_call(n16, st16, wseg, wblk, xs, z_ds, w1k_pad,
                     params["val_main"], pk, params["card_W"],
                     params["card_b"])
    return out.reshape(NB, DIM)
